# baseline (speedup 1.0000x reference)
import functools

import jax
import jax.numpy as jnp
from jax import lax
from jax.experimental import pallas as pl
from jax.experimental.pallas import tpu as pltpu
from jax.experimental.pallas import tpu_sc as plsc

SEQ = 8192
D_IN = 64
REP = 16
D_OUT = D_IN * REP
LANES = 16

NUM_CORES = 2
NUM_SUBCORES = 16
NW = NUM_CORES * NUM_SUBCORES
ROWS_W = SEQ // NW
CHUNK = 16
NCHUNK = ROWS_W // CHUNK


def _expand_body(table_hbm, out_hbm, in0, in1, out0, out1,
                 isem0, isem1, osem0, osem1):
    wid = lax.axis_index("s") * NUM_CORES + lax.axis_index("c")
    base = wid * ROWS_W

    bcast_idx = [jnp.full((LANES,), g, jnp.int32) for g in range(LANES)]

    def bcast(v, idx):
        return jnp.take_along_axis(v, idx, axis=0, mode="promise_in_bounds")

    def fill(in_ref, out_ref):
        def row_body(r, carry):
            src_row = in_ref.at[r]
            dst_row = out_ref.at[r]
            for q in range(D_IN // LANES):
                v = src_row[pl.ds(q * LANES, LANES)]
                for g in range(LANES):
                    k = q * LANES + g
                    dst_row[pl.ds(k * REP, REP)] = bcast(v, bcast_idx[g])
            return carry
        lax.fori_loop(0, CHUNK, row_body, 0)

    in_bufs = (in0, in1)
    out_bufs = (out0, out1)
    isems = (isem0, isem1)
    osems = (osem0, osem1)

    def stage_in(c, b):
        return pltpu.async_copy(
            table_hbm.at[pl.ds(base + c * CHUNK, CHUNK)], in_bufs[b], isems[b])

    in_copies = [stage_in(0, 0), None]
    out_copies = [None, None]
    for c in range(NCHUNK):
        b = c % 2
        nb = (c + 1) % 2
        if c + 1 < NCHUNK:
            in_copies[nb] = stage_in(c + 1, nb)
        in_copies[b].wait()
        if out_copies[b] is not None:
            out_copies[b].wait()
        fill(in_bufs[b], out_bufs[b])
        out_copies[b] = pltpu.async_copy(
            out_bufs[b],
            out_hbm.at[pl.ds(base + c * CHUNK, CHUNK)],
            osems[b])
    for b in range(2):
        if out_copies[b] is not None:
            out_copies[b].wait()


@jax.jit
def _expand(pos_table):
    mesh = plsc.VectorSubcoreMesh(core_axis_name="c", subcore_axis_name="s")
    return pl.kernel(
        _expand_body,
        mesh=mesh,
        out_type=jax.ShapeDtypeStruct((SEQ, D_OUT), jnp.float32),
        scratch_types=[
            pltpu.VMEM((CHUNK, D_IN), jnp.float32),
            pltpu.VMEM((CHUNK, D_IN), jnp.float32),
            pltpu.VMEM((CHUNK, D_OUT), jnp.float32),
            pltpu.VMEM((CHUNK, D_OUT), jnp.float32),
            pltpu.SemaphoreType.DMA,
            pltpu.SemaphoreType.DMA,
            pltpu.SemaphoreType.DMA,
            pltpu.SemaphoreType.DMA,
        ],
    )(pos_table)


def kernel(inputs, pos_table):
    del inputs
    return _expand(pos_table)


# speedup vs baseline: 1.0808x; 1.0808x over previous
import functools

import jax
import jax.numpy as jnp
from jax import lax
from jax.experimental import pallas as pl
from jax.experimental.pallas import tpu as pltpu
from jax.experimental.pallas import tpu_sc as plsc

SEQ = 8192
D_IN = 64
REP = 16
D_OUT = D_IN * REP
LANES = 16

NUM_CORES = 2
NUM_SUBCORES = 16
NW = NUM_CORES * NUM_SUBCORES
ROWS_W = SEQ // NW
CHUNK = 32
NCHUNK = ROWS_W // CHUNK


def _expand_body(table_hbm, out_hbm, in0, in1, out0, out1,
                 isem0, isem1, osem0, osem1):
    wid = lax.axis_index("s") * NUM_CORES + lax.axis_index("c")
    base = wid * ROWS_W

    bcast_idx = [jnp.full((LANES,), g, jnp.int32) for g in range(LANES)]

    def bcast(v, idx):
        return jnp.take_along_axis(v, idx, axis=0, mode="promise_in_bounds")

    def fill(in_ref, out_ref):
        def row_body(r, carry):
            src_row = in_ref.at[r]
            dst_row = out_ref.at[r]
            for q in range(D_IN // LANES):
                v = src_row[pl.ds(q * LANES, LANES)]
                for g in range(LANES):
                    k = q * LANES + g
                    dst_row[pl.ds(k * REP, REP)] = bcast(v, bcast_idx[g])
            return carry
        lax.fori_loop(0, CHUNK, row_body, 0)

    in_bufs = (in0, in1)
    out_bufs = (out0, out1)
    isems = (isem0, isem1)
    osems = (osem0, osem1)

    def stage_in(c, b):
        return pltpu.async_copy(
            table_hbm.at[pl.ds(base + c * CHUNK, CHUNK)], in_bufs[b], isems[b])

    in_copies = [stage_in(0, 0), None]
    out_copies = [None, None]
    for c in range(NCHUNK):
        b = c % 2
        nb = (c + 1) % 2
        if c + 1 < NCHUNK:
            in_copies[nb] = stage_in(c + 1, nb)
        in_copies[b].wait()
        if out_copies[b] is not None:
            out_copies[b].wait()
        fill(in_bufs[b], out_bufs[b])
        out_copies[b] = pltpu.async_copy(
            out_bufs[b],
            out_hbm.at[pl.ds(base + c * CHUNK, CHUNK)],
            osems[b])
    for b in range(2):
        if out_copies[b] is not None:
            out_copies[b].wait()


@jax.jit
def _expand(pos_table):
    mesh = plsc.VectorSubcoreMesh(core_axis_name="c", subcore_axis_name="s")
    return pl.kernel(
        _expand_body,
        mesh=mesh,
        compiler_params=pltpu.CompilerParams(
            disable_bounds_checks=True, skip_device_barrier=True),
        out_type=jax.ShapeDtypeStruct((SEQ, D_OUT), jnp.float32),
        scratch_types=[
            pltpu.VMEM((CHUNK, D_IN), jnp.float32),
            pltpu.VMEM((CHUNK, D_IN), jnp.float32),
            pltpu.VMEM((CHUNK, D_OUT), jnp.float32),
            pltpu.VMEM((CHUNK, D_OUT), jnp.float32),
            pltpu.SemaphoreType.DMA,
            pltpu.SemaphoreType.DMA,
            pltpu.SemaphoreType.DMA,
            pltpu.SemaphoreType.DMA,
        ],
    )(pos_table)


def kernel(inputs, pos_table):
    del inputs
    return _expand(pos_table)
